# baseline (speedup 1.0000x reference)
import functools

import jax
import jax.numpy as jnp
from jax import lax
from jax.experimental import pallas as pl
from jax.experimental.pallas import tpu as pltpu
from jax.experimental.pallas import tpu_sc as plsc

B, T, D = 1, 2048, 1024
H = 1408
E = 8
K = 2
N = B * T
A = N * K
BA = 256
NB = A // BA + E
P = NB * BA
BT = 256

NC, NS = 2, 16
NW = NC * NS
RPW = P // NW
GCH = 96
TPW = N // NW
CT = 32


def _silu(v):
    return v * jax.nn.sigmoid(v)


def _mm(a, b):
    return jax.lax.dot(a.astype(jnp.bfloat16), b.astype(jnp.bfloat16),
                       preferred_element_type=jnp.float32)


def _router_body(x_ref, wr_ref, tp_ref, ti_ref, xb16_ref):
    xb = x_ref[...]
    xb16_ref[...] = xb.astype(jnp.bfloat16)
    logits = xb @ wr_ref[...]
    mx = jnp.max(logits, axis=-1, keepdims=True)
    ex = jnp.exp(logits - mx)
    probs = ex / jnp.sum(ex, axis=-1, keepdims=True)
    ii = jax.lax.broadcasted_iota(jnp.int32, probs.shape, 1)
    m1 = jnp.max(probs, axis=-1, keepdims=True)
    i1 = jnp.min(jnp.where(probs == m1, ii, E), axis=-1, keepdims=True)
    p2 = jnp.where(ii == i1, -1.0, probs)
    m2 = jnp.max(p2, axis=-1, keepdims=True)
    i2 = jnp.min(jnp.where(p2 == m2, ii, E), axis=-1, keepdims=True)
    s = m1 + m2 + 1e-9
    tp_ref[...] = jnp.concatenate([m1 / s, m2 / s], axis=1)
    ti_ref[...] = jnp.concatenate([i1, i2], axis=1)


def _router(flat, Wr):
    return pl.pallas_call(
        _router_body,
        grid=(N // BT,),
        in_specs=[
            pl.BlockSpec((BT, D), lambda t: (t, 0)),
            pl.BlockSpec((D, E), lambda t: (0, 0)),
        ],
        out_specs=[
            pl.BlockSpec((BT, K), lambda t: (t, 0)),
            pl.BlockSpec((BT, K), lambda t: (t, 0)),
            pl.BlockSpec((BT, D), lambda t: (t, 0)),
        ],
        out_shape=[
            jax.ShapeDtypeStruct((N, K), jnp.float32),
            jax.ShapeDtypeStruct((N, K), jnp.int32),
            jax.ShapeDtypeStruct((N, D), jnp.bfloat16),
        ],
    )(flat, Wr)


def _shared_body(x_ref, wgs_ref, wus_ref, wds_ref, sh_ref):
    xb = x_ref[...]
    sh = _silu(_mm(xb, wgs_ref[...])) * _mm(xb, wus_ref[...])
    sh_ref[...] = _mm(sh, wds_ref[...])


def _shared(flat, Wg_s, Wu_s, Wd_s):
    return pl.pallas_call(
        _shared_body,
        grid=(N // BT,),
        in_specs=[
            pl.BlockSpec((BT, D), lambda t: (t, 0)),
            pl.BlockSpec((D, H), lambda t: (0, 0)),
            pl.BlockSpec((D, H), lambda t: (0, 0)),
            pl.BlockSpec((H, D), lambda t: (0, 0)),
        ],
        out_specs=pl.BlockSpec((BT, D), lambda t: (t, 0)),
        out_shape=jax.ShapeDtypeStruct((N, D), jnp.float32),
    )(flat, Wg_s, Wu_s, Wd_s)


def _metadata(tp, ti):
    e_a = ti.reshape(A)
    w_a = tp.reshape(A)
    toks = jnp.arange(A, dtype=jnp.int32) // K
    oh = (e_a[:, None] == jnp.arange(E, dtype=jnp.int32)).astype(jnp.int32)
    csum = jnp.cumsum(oh, axis=0)
    counts = csum[-1]
    rank = jnp.sum((csum - 1) * oh, axis=1)
    pc = (counts + BA - 1) // BA
    cum_pc = jnp.cumsum(pc)
    bstart = jnp.concatenate([jnp.zeros(1, jnp.int32), cum_pc[:-1]])
    pos = bstart[e_a] * BA + rank
    tok_arr = jnp.zeros(P, jnp.int32).at[pos].set(toks)
    wgt_arr = jnp.zeros(P, jnp.float32).at[pos].set(w_a)
    nb_used = cum_pc[-1:]
    bi = jnp.arange(NB, dtype=jnp.int32)
    blk_exp = jnp.minimum(
        jnp.sum((bi[:, None] >= cum_pc[None, :]).astype(jnp.int32), axis=1),
        E - 1)
    pos2 = pos.reshape(N, K)
    return tok_arr, wgt_arr, blk_exp, nb_used, pos2[:, 0], pos2[:, 1]


def _gather_body(tok_hbm, x_hbm, out_hbm, i0_v, i1_v, r0_v, r1_v, sem):
    wid = lax.axis_index("s") * NC + lax.axis_index("c")
    base = wid * RPW
    pltpu.sync_copy(tok_hbm.at[pl.ds(base, GCH)], i0_v)
    cp0 = pltpu.async_copy(x_hbm.at[i0_v], r0_v, sem)
    pltpu.sync_copy(tok_hbm.at[pl.ds(base + GCH, GCH)], i1_v)
    cp1 = pltpu.async_copy(x_hbm.at[i1_v], r1_v, sem)
    cp0.wait()
    pltpu.sync_copy(r0_v, out_hbm.at[pl.ds(base, GCH)])
    cp1.wait()
    pltpu.sync_copy(r1_v, out_hbm.at[pl.ds(base + GCH, GCH)])


def _gather(tok_arr, flat_i32):
    mesh = plsc.VectorSubcoreMesh(core_axis_name="c", subcore_axis_name="s")
    f = functools.partial(
        pl.kernel,
        mesh=mesh,
        out_type=jax.ShapeDtypeStruct((P, D // 2), jnp.int32),
        scratch_types=[
            pltpu.VMEM((GCH,), jnp.int32),
            pltpu.VMEM((GCH,), jnp.int32),
            pltpu.VMEM((GCH, D // 2), jnp.int32),
            pltpu.VMEM((GCH, D // 2), jnp.int32),
            pltpu.SemaphoreType.DMA,
        ],
    )(_gather_body)
    return f(tok_arr, flat_i32)


def _ffn_body(be_ref, nb_ref, xs_ref, wgt_ref, wg_ref, wu_ref, wd_ref, ys_ref):
    i = pl.program_id(0)

    @pl.when(i < nb_ref[0])
    def _():
        xb = xs_ref[...]
        hg = _mm(xb, wg_ref[0])
        hu = _mm(xb, wu_ref[0])
        y = _mm(_silu(hg) * hu, wd_ref[0])
        ys_ref[...] = y * wgt_ref[0, 0, :][:, None]


def _grouped_ffn(blk_exp, nb_used, xs, wgt_arr, Wg, Wu, Wd):
    grid_spec = pltpu.PrefetchScalarGridSpec(
        num_scalar_prefetch=2,
        grid=(NB,),
        in_specs=[
            pl.BlockSpec((BA, D), lambda i, be, nb: (i, 0)),
            pl.BlockSpec((1, 1, BA), lambda i, be, nb: (i, 0, 0)),
            pl.BlockSpec((1, D, H), lambda i, be, nb: (be[i], 0, 0)),
            pl.BlockSpec((1, D, H), lambda i, be, nb: (be[i], 0, 0)),
            pl.BlockSpec((1, H, D), lambda i, be, nb: (be[i], 0, 0)),
        ],
        out_specs=pl.BlockSpec((BA, D), lambda i, be, nb: (i, 0)),
    )
    return pl.pallas_call(
        _ffn_body,
        grid_spec=grid_spec,
        out_shape=jax.ShapeDtypeStruct((P, D), jnp.float32),
        compiler_params=pltpu.CompilerParams(
            dimension_semantics=("arbitrary",),
        ),
    )(blk_exp, nb_used, xs, wgt_arr.reshape(NB, 1, BA), Wg, Wu, Wd)


def _combine_body(p0_hbm, p1_hbm, sh_hbm, ys_hbm, out_hbm,
                  i0_v, i1_v, a_v, b_v, s_v, sem):
    wid = lax.axis_index("s") * NC + lax.axis_index("c")
    base = wid * TPW
    for c in range(TPW // CT):
        tb = base + c * CT
        pltpu.sync_copy(p0_hbm.at[pl.ds(tb, CT)], i0_v)
        pltpu.sync_copy(p1_hbm.at[pl.ds(tb, CT)], i1_v)
        pltpu.async_copy(ys_hbm.at[i0_v], a_v, sem).wait()
        pltpu.async_copy(ys_hbm.at[i1_v], b_v, sem).wait()
        pltpu.sync_copy(sh_hbm.at[pl.ds(tb, CT)], s_v)

        def _row(r, _):
            def _vec(j, _):
                sl = pl.ds(j * 16, 16)
                s_v[r, sl] = s_v[r, sl] + a_v[r, sl] + b_v[r, sl]
                return 0
            return lax.fori_loop(0, D // 16, _vec, 0, unroll=4)

        lax.fori_loop(0, CT, _row, 0)
        pltpu.sync_copy(s_v, out_hbm.at[pl.ds(tb, CT)])


def _combine(p0, p1, shared_out, ys):
    mesh = plsc.VectorSubcoreMesh(core_axis_name="c", subcore_axis_name="s")
    f = functools.partial(
        pl.kernel,
        mesh=mesh,
        out_type=jax.ShapeDtypeStruct((N, D), jnp.float32),
        scratch_types=[
            pltpu.VMEM((CT,), jnp.int32),
            pltpu.VMEM((CT,), jnp.int32),
            pltpu.VMEM((CT, D), jnp.float32),
            pltpu.VMEM((CT, D), jnp.float32),
            pltpu.VMEM((CT, D), jnp.float32),
            pltpu.SemaphoreType.DMA,
        ],
    )(_combine_body)
    return f(p0, p1, shared_out, ys)


def kernel(x, Wg_s, Wu_s, Wd_s, Wr, Wg, Wu, Wd):
    flat = x.reshape(N, D)
    tp, ti, flat16 = _router(flat, Wr)
    tok_arr, wgt_arr, blk_exp, nb_used, p0, p1 = _metadata(tp, ti)
    flat_i32 = lax.bitcast_convert_type(flat16.reshape(N, D // 2, 2),
                                        jnp.int32)
    xs_i32 = _gather(tok_arr, flat_i32)
    shared_out = _shared(flat, Wg_s, Wu_s, Wd_s)
    xs = lax.bitcast_convert_type(xs_i32, jnp.bfloat16).reshape(P, D)
    ys = _grouped_ffn(blk_exp, nb_used, xs, wgt_arr, Wg, Wu, Wd)
    out = _combine(p0, p1, shared_out, ys)
    return out.reshape(x.shape)


# speedup vs baseline: 1.6476x; 1.6476x over previous
import functools

import jax
import jax.numpy as jnp
from jax import lax
from jax.experimental import pallas as pl
from jax.experimental.pallas import tpu as pltpu
from jax.experimental.pallas import tpu_sc as plsc

B, T, D = 1, 2048, 1024
H = 1408
E = 8
K = 2
N = B * T
A = N * K
BA = 256
NB = A // BA + E
P = NB * BA
BT = 256

NC, NS = 2, 16
NW = NC * NS
RPW = P // NW
GCH = 96
TPW = N // NW
CT = 32


def _silu(v):
    return v * jax.nn.sigmoid(v)


def _mm(a, b):
    return jax.lax.dot(a.astype(jnp.bfloat16), b.astype(jnp.bfloat16),
                       preferred_element_type=jnp.float32)


def _router_body(x_ref, wr_ref, tp_ref, ti_ref, xi_ref):
    xb = x_ref[...]
    bi = lax.bitcast_convert_type(xb, jnp.int32)
    rnd = (bi + 0x7FFF + ((bi >> 16) & 1)) >> 16
    xi_ref[...] = (rnd[:, :D // 2] & 0xFFFF) | (rnd[:, D // 2:] << 16)
    logits = xb @ wr_ref[...]
    mx = jnp.max(logits, axis=-1, keepdims=True)
    ex = jnp.exp(logits - mx)
    probs = ex / jnp.sum(ex, axis=-1, keepdims=True)
    ii = jax.lax.broadcasted_iota(jnp.int32, probs.shape, 1)
    m1 = jnp.max(probs, axis=-1, keepdims=True)
    i1 = jnp.min(jnp.where(probs == m1, ii, E), axis=-1, keepdims=True)
    p2 = jnp.where(ii == i1, -1.0, probs)
    m2 = jnp.max(p2, axis=-1, keepdims=True)
    i2 = jnp.min(jnp.where(p2 == m2, ii, E), axis=-1, keepdims=True)
    s = m1 + m2 + 1e-9
    tp_ref[...] = jnp.concatenate([m1 / s, m2 / s], axis=1)
    ti_ref[...] = jnp.concatenate([i1, i2], axis=1)


def _router(flat, Wr):
    return pl.pallas_call(
        _router_body,
        grid=(N // BT,),
        in_specs=[
            pl.BlockSpec((BT, D), lambda t: (t, 0)),
            pl.BlockSpec((D, E), lambda t: (0, 0)),
        ],
        out_specs=[
            pl.BlockSpec((BT, K), lambda t: (t, 0)),
            pl.BlockSpec((BT, K), lambda t: (t, 0)),
            pl.BlockSpec((BT, D // 2), lambda t: (t, 0)),
        ],
        out_shape=[
            jax.ShapeDtypeStruct((N, K), jnp.float32),
            jax.ShapeDtypeStruct((N, K), jnp.int32),
            jax.ShapeDtypeStruct((N, D // 2), jnp.int32),
        ],
    )(flat, Wr)


def _shared_body(x_ref, wgs_ref, wus_ref, wds_ref, sh_ref):
    xb = x_ref[...]
    sh = _silu(_mm(xb, wgs_ref[...])) * _mm(xb, wus_ref[...])
    sh_ref[...] = _mm(sh, wds_ref[...])


def _shared(flat, Wg_s, Wu_s, Wd_s):
    return pl.pallas_call(
        _shared_body,
        grid=(N // BT,),
        in_specs=[
            pl.BlockSpec((BT, D), lambda t: (t, 0)),
            pl.BlockSpec((D, H), lambda t: (0, 0)),
            pl.BlockSpec((D, H), lambda t: (0, 0)),
            pl.BlockSpec((H, D), lambda t: (0, 0)),
        ],
        out_specs=pl.BlockSpec((BT, D), lambda t: (t, 0)),
        out_shape=jax.ShapeDtypeStruct((N, D), jnp.float32),
    )(flat, Wg_s, Wu_s, Wd_s)


def _metadata(tp, ti):
    e_a = ti.reshape(A)
    w_a = tp.reshape(A)
    toks = jnp.arange(A, dtype=jnp.int32) // K
    oh = (e_a[:, None] == jnp.arange(E, dtype=jnp.int32)).astype(jnp.int32)
    csum = jnp.cumsum(oh, axis=0)
    counts = csum[-1]
    rank = jnp.sum((csum - 1) * oh, axis=1)
    pc = (counts + BA - 1) // BA
    cum_pc = jnp.cumsum(pc)
    bstart = jnp.concatenate([jnp.zeros(1, jnp.int32), cum_pc[:-1]])
    pos = bstart[e_a] * BA + rank
    tok_arr = jnp.zeros(P, jnp.int32).at[pos].set(toks)
    wgt_arr = jnp.zeros(P, jnp.float32).at[pos].set(w_a)
    nb_used = cum_pc[-1:]
    bi = jnp.arange(NB, dtype=jnp.int32)
    blk_exp = jnp.minimum(
        jnp.sum((bi[:, None] >= cum_pc[None, :]).astype(jnp.int32), axis=1),
        E - 1)
    pos2 = pos.reshape(N, K)
    return tok_arr, wgt_arr, blk_exp, nb_used, pos2[:, 0], pos2[:, 1]


GNS = 4
GCR = RPW // GNS


def _gather_body(tok_hbm, x_hbm, out_hbm, *args):
    idx_vs = args[:GNS]
    row_vs = args[GNS:2 * GNS]
    sem = args[2 * GNS]
    wid = lax.axis_index("s") * NC + lax.axis_index("c")
    base = wid * RPW
    cps = []
    for ch in range(GNS):
        pltpu.sync_copy(tok_hbm.at[pl.ds(base + ch * GCR, GCR)], idx_vs[ch])
        cps.append(pltpu.async_copy(x_hbm.at[idx_vs[ch]], row_vs[ch], sem))
    for ch in range(GNS):
        cps[ch].wait()
        pltpu.sync_copy(row_vs[ch], out_hbm.at[pl.ds(base + ch * GCR, GCR)])


def _gather(tok_arr, flat_i32):
    mesh = plsc.VectorSubcoreMesh(core_axis_name="c", subcore_axis_name="s")
    f = functools.partial(
        pl.kernel,
        mesh=mesh,
        out_type=jax.ShapeDtypeStruct((P, D // 2), jnp.int32),
        scratch_types=(
            [pltpu.VMEM((GCR,), jnp.int32) for _ in range(GNS)]
            + [pltpu.VMEM((GCR, D // 2), jnp.int32) for _ in range(GNS)]
            + [pltpu.SemaphoreType.DMA]
        ),
    )(_gather_body)
    return f(tok_arr, flat_i32)


def _ffn_body(be_ref, nb_ref, xs_ref, wgt_ref, wg_ref, wu_ref, wd_ref, ys_ref):
    i = pl.program_id(0)

    @pl.when(i < nb_ref[0])
    def _():
        xi = xs_ref[...]
        xlo = lax.bitcast_convert_type(xi << 16, jnp.float32)
        xhi = lax.bitcast_convert_type(xi & jnp.int32(-65536),
                                       jnp.float32)
        wg = wg_ref[0]
        wu = wu_ref[0]
        hg = _mm(xlo, wg[:D // 2]) + _mm(xhi, wg[D // 2:])
        hu = _mm(xlo, wu[:D // 2]) + _mm(xhi, wu[D // 2:])
        y = _mm(_silu(hg) * hu, wd_ref[0])
        ys_ref[...] = y * wgt_ref[0, 0, :][:, None]


def _grouped_ffn(blk_exp, nb_used, xs, wgt_arr, Wg, Wu, Wd):
    grid_spec = pltpu.PrefetchScalarGridSpec(
        num_scalar_prefetch=2,
        grid=(NB,),
        in_specs=[
            pl.BlockSpec((BA, D // 2), lambda i, be, nb: (i, 0)),
            pl.BlockSpec((1, 1, BA), lambda i, be, nb: (i, 0, 0)),
            pl.BlockSpec((1, D, H), lambda i, be, nb: (be[i], 0, 0)),
            pl.BlockSpec((1, D, H), lambda i, be, nb: (be[i], 0, 0)),
            pl.BlockSpec((1, H, D), lambda i, be, nb: (be[i], 0, 0)),
        ],
        out_specs=pl.BlockSpec((BA, D), lambda i, be, nb: (i, 0)),
    )
    return pl.pallas_call(
        _ffn_body,
        grid_spec=grid_spec,
        out_shape=jax.ShapeDtypeStruct((P, D), jnp.float32),
        compiler_params=pltpu.CompilerParams(
            dimension_semantics=("arbitrary",),
        ),
    )(blk_exp, nb_used, xs, wgt_arr.reshape(NB, 1, BA), Wg, Wu, Wd)


def _combine_body(p0_hbm, p1_hbm, sh_hbm, ys_hbm, out_hbm,
                  i0_v, i1_v, a_v, b_v, s_v, sem):
    wid = lax.axis_index("s") * NC + lax.axis_index("c")
    base = wid * TPW
    for c in range(TPW // CT):
        tb = base + c * CT
        pltpu.sync_copy(p0_hbm.at[pl.ds(tb, CT)], i0_v)
        pltpu.sync_copy(p1_hbm.at[pl.ds(tb, CT)], i1_v)
        pltpu.async_copy(ys_hbm.at[i0_v], a_v, sem).wait()
        pltpu.async_copy(ys_hbm.at[i1_v], b_v, sem).wait()
        pltpu.sync_copy(sh_hbm.at[pl.ds(tb, CT)], s_v)

        def _row(r, _):
            def _vec(j, _):
                sl = pl.ds(j * 16, 16)
                s_v[r, sl] = s_v[r, sl] + a_v[r, sl] + b_v[r, sl]
                return 0
            return lax.fori_loop(0, D // 16, _vec, 0, unroll=4)

        lax.fori_loop(0, CT, _row, 0)
        pltpu.sync_copy(s_v, out_hbm.at[pl.ds(tb, CT)])


def _combine(p0, p1, shared_out, ys):
    mesh = plsc.VectorSubcoreMesh(core_axis_name="c", subcore_axis_name="s")
    f = functools.partial(
        pl.kernel,
        mesh=mesh,
        out_type=jax.ShapeDtypeStruct((N, D), jnp.float32),
        scratch_types=[
            pltpu.VMEM((CT,), jnp.int32),
            pltpu.VMEM((CT,), jnp.int32),
            pltpu.VMEM((CT, D), jnp.float32),
            pltpu.VMEM((CT, D), jnp.float32),
            pltpu.VMEM((CT, D), jnp.float32),
            pltpu.SemaphoreType.DMA,
        ],
    )(_combine_body)
    return f(p0, p1, shared_out, ys)


def kernel(x, Wg_s, Wu_s, Wd_s, Wr, Wg, Wu, Wd):
    flat = x.reshape(N, D)
    tp, ti, xi32 = _router(flat, Wr)
    tok_arr, wgt_arr, blk_exp, nb_used, p0, p1 = _metadata(tp, ti)
    xs_i32 = _gather(tok_arr, xi32)
    shared_out = _shared(flat, Wg_s, Wu_s, Wd_s)
    ys = _grouped_ffn(blk_exp, nb_used, xs_i32, wgt_arr, Wg, Wu, Wd)
    out = _combine(p0, p1, shared_out, ys)
    return out.reshape(x.shape)


# speedup vs baseline: 2.3337x; 1.4164x over previous
import functools

import jax
import jax.numpy as jnp
from jax import lax
from jax.experimental import pallas as pl
from jax.experimental.pallas import tpu as pltpu
from jax.experimental.pallas import tpu_sc as plsc

B, T, D = 1, 2048, 1024
H = 1408
E = 8
K = 2
N = B * T
A = N * K
BA = 256
NB = A // BA + E
P = NB * BA
BT = 256
A2 = BT * K

NC, NS = 2, 16
NW = NC * NS
TPW = N // NW
CT = 32


def _silu(v):
    return v * jax.nn.sigmoid(v)


def _mm(a, b):
    return jax.lax.dot(a.astype(jnp.bfloat16), b.astype(jnp.bfloat16),
                       preferred_element_type=jnp.float32)


def _router_body(x_ref, wr_ref, tp_ref, ti_ref, rk_ref, cnt_ref, xb16_ref,
                 cnt_scr):
    t = pl.program_id(0)
    xb = x_ref[...]
    xb16_ref[...] = xb.astype(jnp.bfloat16)
    logits = xb @ wr_ref[...]
    mx = jnp.max(logits, axis=-1, keepdims=True)
    ex = jnp.exp(logits - mx)
    probs = ex / jnp.sum(ex, axis=-1, keepdims=True)
    ii = jax.lax.broadcasted_iota(jnp.int32, probs.shape, 1)
    m1 = jnp.max(probs, axis=-1, keepdims=True)
    i1 = jnp.min(jnp.where(probs == m1, ii, E), axis=-1, keepdims=True)
    p2 = jnp.where(ii == i1, -1.0, probs)
    m2 = jnp.max(p2, axis=-1, keepdims=True)
    i2 = jnp.min(jnp.where(p2 == m2, ii, E), axis=-1, keepdims=True)
    s = m1 + m2 + 1e-9
    tp_ref[...] = jnp.concatenate([m1 / s, m2 / s], axis=1)
    ti_ref[...] = jnp.concatenate([i1, i2], axis=1)

    @pl.when(t == 0)
    def _init():
        cnt_scr[...] = jnp.zeros((1, E), jnp.float32)

    ie = jax.lax.broadcasted_iota(jnp.int32, (BT, E), 1)
    oh0 = (i1 == ie).astype(jnp.float32)
    oh1 = (i2 == ie).astype(jnp.float32)
    ra = jax.lax.broadcasted_iota(jnp.int32, (BT, BT), 0)
    rb = jax.lax.broadcasted_iota(jnp.int32, (BT, BT), 1)
    stril = (ra > rb).astype(jnp.float32)
    cnt = cnt_scr[...]
    tot0 = jnp.sum(oh0, axis=0, keepdims=True)
    r0 = (jnp.sum(jax.lax.dot(stril, oh0,
                              preferred_element_type=jnp.float32) * oh0,
                  axis=1, keepdims=True)
          + jnp.sum(cnt * oh0, axis=1, keepdims=True))
    r1 = (jnp.sum(jax.lax.dot(stril, oh1,
                              preferred_element_type=jnp.float32) * oh1,
                  axis=1, keepdims=True)
          + jnp.sum((cnt + tot0) * oh1, axis=1, keepdims=True))
    rk_ref[...] = jnp.concatenate([r0, r1], axis=1).astype(jnp.int32)
    cnt_scr[...] = cnt + tot0 + jnp.sum(oh1, axis=0, keepdims=True)
    cnt_ref[...] = cnt_scr[...]


def _router(flat, Wr):
    return pl.pallas_call(
        _router_body,
        grid=(N // BT,),
        in_specs=[
            pl.BlockSpec((BT, D), lambda t: (t, 0)),
            pl.BlockSpec((D, E), lambda t: (0, 0)),
        ],
        out_specs=[
            pl.BlockSpec((BT, K), lambda t: (t, 0)),
            pl.BlockSpec((BT, K), lambda t: (t, 0)),
            pl.BlockSpec((BT, K), lambda t: (t, 0)),
            pl.BlockSpec((1, E), lambda t: (0, 0)),
            pl.BlockSpec((BT, D), lambda t: (t, 0)),
        ],
        out_shape=[
            jax.ShapeDtypeStruct((N, K), jnp.float32),
            jax.ShapeDtypeStruct((N, K), jnp.int32),
            jax.ShapeDtypeStruct((N, K), jnp.int32),
            jax.ShapeDtypeStruct((1, E), jnp.float32),
            jax.ShapeDtypeStruct((N, D), jnp.bfloat16),
        ],
        scratch_shapes=[pltpu.VMEM((1, E), jnp.float32)],
        compiler_params=pltpu.CompilerParams(
            dimension_semantics=("arbitrary",),
        ),
    )(flat, Wr)


def _shared_body(x_ref, wgs_ref, wus_ref, wds_ref, sh_ref):
    xb = x_ref[...]
    sh = _silu(_mm(xb, wgs_ref[...])) * _mm(xb, wus_ref[...])
    sh_ref[...] = _mm(sh, wds_ref[...])


def _shared(flat, Wg_s, Wu_s, Wd_s):
    return pl.pallas_call(
        _shared_body,
        grid=(N // BT,),
        in_specs=[
            pl.BlockSpec((BT, D), lambda t: (t, 0)),
            pl.BlockSpec((D, H), lambda t: (0, 0)),
            pl.BlockSpec((D, H), lambda t: (0, 0)),
            pl.BlockSpec((H, D), lambda t: (0, 0)),
        ],
        out_specs=pl.BlockSpec((BT, D), lambda t: (t, 0)),
        out_shape=jax.ShapeDtypeStruct((N, D), jnp.float32),
    )(flat, Wg_s, Wu_s, Wd_s)


def _metadata(ti, rk, counts):
    pc = (counts + BA - 1) // BA
    cum_pc = jnp.cumsum(pc)
    bstart = jnp.concatenate([jnp.zeros(1, jnp.int32), cum_pc[:-1]])
    nb_used = cum_pc[-1:]
    bi = jnp.arange(NB, dtype=jnp.int32)
    blk_exp = jnp.minimum(
        jnp.sum((bi[:, None] >= cum_pc[None, :]).astype(jnp.int32), axis=1),
        E - 1)
    rbase = (bi - bstart[blk_exp]) * BA
    pos2 = bstart[ti] * BA + rk
    return blk_exp, rbase, nb_used, pos2[:, 0], pos2[:, 1]


def _ffn_body(be_ref, rb_ref, nb_ref, ti0_ref, ti1_ref, rk0_ref, rk1_ref,
              tp0_ref, tp1_ref, xb_ref, wg_ref, wu_ref, wd_ref, ys_ref):
    i = pl.program_id(0)

    @pl.when(i < nb_ref[0])
    def _():
        e = be_ref[i]
        rb = rb_ref[i]
        rows = jax.lax.broadcasted_iota(jnp.int32, (BA, N), 0) + rb
        c0 = (ti0_ref[...] == e) & (rk0_ref[...] == rows)
        c1 = (ti1_ref[...] == e) & (rk1_ref[...] == rows)
        oh = c0.astype(jnp.bfloat16) + c1.astype(jnp.bfloat16)
        xs = jax.lax.dot(oh, xb_ref[...],
                         preferred_element_type=jnp.float32)
        wgt = jnp.sum(jnp.where(c0, tp0_ref[...], 0.0)
                      + jnp.where(c1, tp1_ref[...], 0.0),
                      axis=1, keepdims=True)
        hg = _mm(xs, wg_ref[0])
        hu = _mm(xs, wu_ref[0])
        y = _mm(_silu(hg) * hu, wd_ref[0])
        ys_ref[...] = y * wgt


def _grouped_ffn(blk_exp, rbase, nb_used, meta_rows, xb16, Wg, Wu, Wd):
    grid_spec = pltpu.PrefetchScalarGridSpec(
        num_scalar_prefetch=3,
        grid=(NB,),
        in_specs=(
            [pl.BlockSpec((1, N), lambda i, be, rb, nb: (0, 0))] * 6
            + [
                pl.BlockSpec((N, D), lambda i, be, rb, nb: (0, 0)),
                pl.BlockSpec((1, D, H), lambda i, be, rb, nb: (be[i], 0, 0)),
                pl.BlockSpec((1, D, H), lambda i, be, rb, nb: (be[i], 0, 0)),
                pl.BlockSpec((1, H, D), lambda i, be, rb, nb: (be[i], 0, 0)),
            ]
        ),
        out_specs=pl.BlockSpec((BA, D), lambda i, be, rb, nb: (i, 0)),
    )
    return pl.pallas_call(
        _ffn_body,
        grid_spec=grid_spec,
        out_shape=jax.ShapeDtypeStruct((P, D), jnp.float32),
        compiler_params=pltpu.CompilerParams(
            dimension_semantics=("arbitrary",),
        ),
    )(blk_exp, rbase, nb_used, *meta_rows, xb16, Wg, Wu, Wd)


def _combine_body(p0_hbm, p1_hbm, sh_hbm, ys_hbm, out_hbm,
                  i0_v, i1_v, a_v, b_v, s_v, sem):
    wid = lax.axis_index("s") * NC + lax.axis_index("c")
    base = wid * TPW
    for c in range(TPW // CT):
        tb = base + c * CT
        pltpu.sync_copy(p0_hbm.at[pl.ds(tb, CT)], i0_v)
        pltpu.sync_copy(p1_hbm.at[pl.ds(tb, CT)], i1_v)
        cpa = pltpu.async_copy(ys_hbm.at[i0_v], a_v, sem)
        cpb = pltpu.async_copy(ys_hbm.at[i1_v], b_v, sem)
        pltpu.sync_copy(sh_hbm.at[pl.ds(tb, CT)], s_v)
        cpa.wait()
        cpb.wait()

        def _row(r, _):
            def _vec(j, _):
                sl = pl.ds(j * 16, 16)
                s_v[r, sl] = s_v[r, sl] + a_v[r, sl] + b_v[r, sl]
                return 0
            return lax.fori_loop(0, D // 16, _vec, 0, unroll=4)

        lax.fori_loop(0, CT, _row, 0)
        pltpu.sync_copy(s_v, out_hbm.at[pl.ds(tb, CT)])


def _combine(p0, p1, shared_out, ys):
    mesh = plsc.VectorSubcoreMesh(core_axis_name="c", subcore_axis_name="s")
    f = functools.partial(
        pl.kernel,
        mesh=mesh,
        out_type=jax.ShapeDtypeStruct((N, D), jnp.float32),
        scratch_types=[
            pltpu.VMEM((CT,), jnp.int32),
            pltpu.VMEM((CT,), jnp.int32),
            pltpu.VMEM((CT, D), jnp.float32),
            pltpu.VMEM((CT, D), jnp.float32),
            pltpu.VMEM((CT, D), jnp.float32),
            pltpu.SemaphoreType.DMA,
        ],
    )(_combine_body)
    return f(p0, p1, shared_out, ys)


def kernel(x, Wg_s, Wu_s, Wd_s, Wr, Wg, Wu, Wd):
    flat = x.reshape(N, D)
    tp, ti, rk, cnts, xb16 = _router(flat, Wr)
    counts = cnts.reshape(E).astype(jnp.int32)
    blk_exp, rbase, nb_used, p0, p1 = _metadata(ti, rk, counts)
    meta_rows = (
        ti[:, 0].reshape(1, N), ti[:, 1].reshape(1, N),
        rk[:, 0].reshape(1, N), rk[:, 1].reshape(1, N),
        tp[:, 0].reshape(1, N), tp[:, 1].reshape(1, N),
    )
    ys = _grouped_ffn(blk_exp, rbase, nb_used, meta_rows, xb16, Wg, Wu, Wd)
    shared_out = _shared(flat, Wg_s, Wu_s, Wd_s)
    out = _combine(p0, p1, shared_out, ys)
    return out.reshape(x.shape)


# speedup vs baseline: 2.4308x; 1.0416x over previous
import functools

import jax
import jax.numpy as jnp
from jax import lax
from jax.experimental import pallas as pl
from jax.experimental.pallas import tpu as pltpu
from jax.experimental.pallas import tpu_sc as plsc

B, T, D = 1, 2048, 1024
H = 1408
E = 8
K = 2
N = B * T
A = N * K
BA = 256
NB = A // BA + E
P = NB * BA
BT = 256
A2 = BT * K

NC, NS = 2, 16
NW = NC * NS
TPW = N // NW
CT = 32


def _silu(v):
    return v * jax.nn.sigmoid(v)


def _mm(a, b):
    return jax.lax.dot(a.astype(jnp.bfloat16), b.astype(jnp.bfloat16),
                       preferred_element_type=jnp.float32)


def _router_body(x_ref, wr_ref, ti0_ref, ti1_ref, rk0_ref, rk1_ref,
                 tp0_ref, tp1_ref, cnt_ref, xb16_ref, cnt_scr):
    t = pl.program_id(0)
    xb = x_ref[...]
    xb16_ref[...] = xb.astype(jnp.bfloat16)
    lt = jax.lax.dot_general(wr_ref[...], xb, (((0,), (1,)), ((), ())),
                             preferred_element_type=jnp.float32)
    mx = jnp.max(lt, axis=0, keepdims=True)
    ex = jnp.exp(lt - mx)
    probs = ex / jnp.sum(ex, axis=0, keepdims=True)
    ie = jax.lax.broadcasted_iota(jnp.int32, (E, BT), 0)
    m1 = jnp.max(probs, axis=0, keepdims=True)
    i1 = jnp.min(jnp.where(probs == m1, ie, E), axis=0, keepdims=True)
    p2 = jnp.where(ie == i1, -1.0, probs)
    m2 = jnp.max(p2, axis=0, keepdims=True)
    i2 = jnp.min(jnp.where(p2 == m2, ie, E), axis=0, keepdims=True)
    s = m1 + m2 + 1e-9
    tp0_ref[...] = m1 / s
    tp1_ref[...] = m2 / s
    ti0_ref[...] = i1
    ti1_ref[...] = i2

    @pl.when(t == 0)
    def _init():
        cnt_scr[...] = jnp.zeros((E, 1), jnp.float32)

    oh0 = (ie == i1).astype(jnp.float32)
    oh1 = (ie == i2).astype(jnp.float32)
    rr = jax.lax.broadcasted_iota(jnp.int32, (BT, BT), 0)
    cc = jax.lax.broadcasted_iota(jnp.int32, (BT, BT), 1)
    striu = (rr < cc).astype(jnp.float32)
    cnt = cnt_scr[...]
    tot0 = jnp.sum(oh0, axis=1, keepdims=True)
    r0 = (jnp.sum(jax.lax.dot(oh0, striu,
                              preferred_element_type=jnp.float32) * oh0,
                  axis=0, keepdims=True)
          + jnp.sum(cnt * oh0, axis=0, keepdims=True))
    r1 = (jnp.sum(jax.lax.dot(oh1, striu,
                              preferred_element_type=jnp.float32) * oh1,
                  axis=0, keepdims=True)
          + jnp.sum((cnt + tot0) * oh1, axis=0, keepdims=True))
    rk0_ref[...] = r0.astype(jnp.int32)
    rk1_ref[...] = r1.astype(jnp.int32)
    cnt_scr[...] = cnt + tot0 + jnp.sum(oh1, axis=1, keepdims=True)
    cnt_ref[...] = cnt_scr[...]


def _router(flat, Wr):
    row_spec = pl.BlockSpec((1, BT), lambda t: (0, t))
    row_shape_i = jax.ShapeDtypeStruct((1, N), jnp.int32)
    row_shape_f = jax.ShapeDtypeStruct((1, N), jnp.float32)
    return pl.pallas_call(
        _router_body,
        grid=(N // BT,),
        in_specs=[
            pl.BlockSpec((BT, D), lambda t: (t, 0)),
            pl.BlockSpec((D, E), lambda t: (0, 0)),
        ],
        out_specs=[
            row_spec, row_spec, row_spec, row_spec, row_spec, row_spec,
            pl.BlockSpec((E, 1), lambda t: (0, 0)),
            pl.BlockSpec((BT, D), lambda t: (t, 0)),
        ],
        out_shape=[
            row_shape_i, row_shape_i, row_shape_i, row_shape_i,
            row_shape_f, row_shape_f,
            jax.ShapeDtypeStruct((E, 1), jnp.float32),
            jax.ShapeDtypeStruct((N, D), jnp.bfloat16),
        ],
        scratch_shapes=[pltpu.VMEM((E, 1), jnp.float32)],
        compiler_params=pltpu.CompilerParams(
            dimension_semantics=("arbitrary",),
        ),
    )(flat, Wr)


def _shared_body(x_ref, wgs_ref, wus_ref, wds_ref, sh_ref):
    xb = x_ref[...]
    sh = _silu(_mm(xb, wgs_ref[...])) * _mm(xb, wus_ref[...])
    sh_ref[...] = _mm(sh, wds_ref[...])


def _shared(flat, Wg_s, Wu_s, Wd_s):
    return pl.pallas_call(
        _shared_body,
        grid=(N // BT,),
        in_specs=[
            pl.BlockSpec((BT, D), lambda t: (t, 0)),
            pl.BlockSpec((D, H), lambda t: (0, 0)),
            pl.BlockSpec((D, H), lambda t: (0, 0)),
            pl.BlockSpec((H, D), lambda t: (0, 0)),
        ],
        out_specs=pl.BlockSpec((BT, D), lambda t: (t, 0)),
        out_shape=jax.ShapeDtypeStruct((N, D), jnp.float32),
    )(flat, Wg_s, Wu_s, Wd_s)


def _metadata(ti0, ti1, rk0, rk1, counts):
    pc = (counts + BA - 1) // BA
    cum_pc = jnp.cumsum(pc)
    bstart = jnp.concatenate([jnp.zeros(1, jnp.int32), cum_pc[:-1]])
    nb_used = cum_pc[-1:]
    bi = jnp.arange(NB, dtype=jnp.int32)
    blk_exp = jnp.minimum(
        jnp.sum((bi[:, None] >= cum_pc[None, :]).astype(jnp.int32), axis=1),
        E - 1)
    rbase = (bi - bstart[blk_exp]) * BA
    p0 = (bstart[ti0] * BA + rk0).reshape(N)
    p1 = (bstart[ti1] * BA + rk1).reshape(N)
    return blk_exp, rbase, nb_used, p0, p1


def _ffn_body(be_ref, rb_ref, nb_ref, ti0_ref, ti1_ref, rk0_ref, rk1_ref,
              tp0_ref, tp1_ref, xb_ref, wg_ref, wu_ref, wd_ref, ys_ref):
    i = pl.program_id(0)

    @pl.when(i < nb_ref[0])
    def _():
        e = be_ref[i]
        rb = rb_ref[i]
        rows = jax.lax.broadcasted_iota(jnp.int32, (BA, N), 0) + rb
        c0 = (ti0_ref[...] == e) & (rk0_ref[...] == rows)
        c1 = (ti1_ref[...] == e) & (rk1_ref[...] == rows)
        oh = c0.astype(jnp.bfloat16) + c1.astype(jnp.bfloat16)
        xs = jax.lax.dot(oh, xb_ref[...],
                         preferred_element_type=jnp.float32)
        wgt = jnp.sum(jnp.where(c0, tp0_ref[...], 0.0)
                      + jnp.where(c1, tp1_ref[...], 0.0),
                      axis=1, keepdims=True)
        hg = _mm(xs, wg_ref[0])
        hu = _mm(xs, wu_ref[0])
        y = _mm(_silu(hg) * hu, wd_ref[0])
        ys_ref[...] = y * wgt


def _grouped_ffn(blk_exp, rbase, nb_used, meta_rows, xb16, Wg, Wu, Wd):
    grid_spec = pltpu.PrefetchScalarGridSpec(
        num_scalar_prefetch=3,
        grid=(NB,),
        in_specs=(
            [pl.BlockSpec((1, N), lambda i, be, rb, nb: (0, 0))] * 6
            + [
                pl.BlockSpec((N, D), lambda i, be, rb, nb: (0, 0)),
                pl.BlockSpec((1, D, H), lambda i, be, rb, nb: (be[i], 0, 0)),
                pl.BlockSpec((1, D, H), lambda i, be, rb, nb: (be[i], 0, 0)),
                pl.BlockSpec((1, H, D), lambda i, be, rb, nb: (be[i], 0, 0)),
            ]
        ),
        out_specs=pl.BlockSpec((BA, D), lambda i, be, rb, nb: (i, 0)),
    )
    return pl.pallas_call(
        _ffn_body,
        grid_spec=grid_spec,
        out_shape=jax.ShapeDtypeStruct((P, D), jnp.float32),
        compiler_params=pltpu.CompilerParams(
            dimension_semantics=("arbitrary",),
        ),
    )(blk_exp, rbase, nb_used, *meta_rows, xb16, Wg, Wu, Wd)


def _combine_body(p0_hbm, p1_hbm, sh_hbm, ys_hbm, out_hbm,
                  i0_v, i1_v, a_v, b_v, s_v, sem):
    wid = lax.axis_index("s") * NC + lax.axis_index("c")
    base = wid * TPW
    for c in range(TPW // CT):
        tb = base + c * CT
        pltpu.sync_copy(p0_hbm.at[pl.ds(tb, CT)], i0_v)
        pltpu.sync_copy(p1_hbm.at[pl.ds(tb, CT)], i1_v)
        cpa = pltpu.async_copy(ys_hbm.at[i0_v], a_v, sem)
        cpb = pltpu.async_copy(ys_hbm.at[i1_v], b_v, sem)
        pltpu.sync_copy(sh_hbm.at[pl.ds(tb, CT)], s_v)
        cpa.wait()
        cpb.wait()

        def _row(r, _):
            def _vec(j, _):
                sl = pl.ds(j * 16, 16)
                s_v[r, sl] = s_v[r, sl] + a_v[r, sl] + b_v[r, sl]
                return 0
            return lax.fori_loop(0, D // 16, _vec, 0, unroll=4)

        lax.fori_loop(0, CT, _row, 0)
        pltpu.sync_copy(s_v, out_hbm.at[pl.ds(tb, CT)])


def _combine(p0, p1, shared_out, ys):
    mesh = plsc.VectorSubcoreMesh(core_axis_name="c", subcore_axis_name="s")
    f = functools.partial(
        pl.kernel,
        mesh=mesh,
        out_type=jax.ShapeDtypeStruct((N, D), jnp.float32),
        scratch_types=[
            pltpu.VMEM((CT,), jnp.int32),
            pltpu.VMEM((CT,), jnp.int32),
            pltpu.VMEM((CT, D), jnp.float32),
            pltpu.VMEM((CT, D), jnp.float32),
            pltpu.VMEM((CT, D), jnp.float32),
            pltpu.SemaphoreType.DMA,
        ],
    )(_combine_body)
    return f(p0, p1, shared_out, ys)


def kernel(x, Wg_s, Wu_s, Wd_s, Wr, Wg, Wu, Wd):
    flat = x.reshape(N, D)
    ti0, ti1, rk0, rk1, tp0, tp1, cnts, xb16 = _router(flat, Wr)
    counts = cnts.reshape(E).astype(jnp.int32)
    blk_exp, rbase, nb_used, p0, p1 = _metadata(ti0, ti1, rk0, rk1, counts)
    meta_rows = (ti0, ti1, rk0, rk1, tp0, tp1)
    ys = _grouped_ffn(blk_exp, rbase, nb_used, meta_rows, xb16, Wg, Wu, Wd)
    shared_out = _shared(flat, Wg_s, Wu_s, Wd_s)
    out = _combine(p0, p1, shared_out, ys)
    return out.reshape(x.shape)


# speedup vs baseline: 2.6886x; 1.1061x over previous
import functools

import jax
import jax.numpy as jnp
from jax import lax
from jax.experimental import pallas as pl
from jax.experimental.pallas import tpu as pltpu
from jax.experimental.pallas import tpu_sc as plsc

B, T, D = 1, 2048, 1024
H = 1408
E = 8
K = 2
N = B * T
A = N * K
BA = 256
NB = A // BA + E
P = NB * BA
BT = 256
A2 = BT * K

NC, NS = 2, 16
NW = NC * NS
TPW = N // NW
CT = 32


def _silu(v):
    return v * jax.nn.sigmoid(v)


def _mm(a, b):
    return jax.lax.dot(a.astype(jnp.bfloat16), b.astype(jnp.bfloat16),
                       preferred_element_type=jnp.float32)


def _router_body(x_ref, wr_ref, wgs_ref, wus_ref, wds_ref,
                 ti0_ref, ti1_ref, rk0_ref, rk1_ref,
                 tp0_ref, tp1_ref, cnt_ref, xb16_ref, sh_ref, cnt_scr):
    t = pl.program_id(0)
    xb = x_ref[...]
    xb16_ref[...] = xb.astype(jnp.bfloat16)
    shh = _silu(_mm(xb, wgs_ref[...])) * _mm(xb, wus_ref[...])
    sh_ref[...] = _mm(shh, wds_ref[...])
    lt = jax.lax.dot_general(wr_ref[...], xb, (((0,), (1,)), ((), ())),
                             preferred_element_type=jnp.float32)
    mx = jnp.max(lt, axis=0, keepdims=True)
    ex = jnp.exp(lt - mx)
    probs = ex / jnp.sum(ex, axis=0, keepdims=True)
    ie = jax.lax.broadcasted_iota(jnp.int32, (E, BT), 0)
    m1 = jnp.max(probs, axis=0, keepdims=True)
    i1 = jnp.min(jnp.where(probs == m1, ie, E), axis=0, keepdims=True)
    p2 = jnp.where(ie == i1, -1.0, probs)
    m2 = jnp.max(p2, axis=0, keepdims=True)
    i2 = jnp.min(jnp.where(p2 == m2, ie, E), axis=0, keepdims=True)
    s = m1 + m2 + 1e-9
    tp0_ref[...] = m1 / s
    tp1_ref[...] = m2 / s
    ti0_ref[...] = i1
    ti1_ref[...] = i2

    @pl.when(t == 0)
    def _init():
        cnt_scr[...] = jnp.zeros((E, 1), jnp.float32)

    oh0 = (ie == i1).astype(jnp.float32)
    oh1 = (ie == i2).astype(jnp.float32)
    rr = jax.lax.broadcasted_iota(jnp.int32, (BT, BT), 0)
    cc = jax.lax.broadcasted_iota(jnp.int32, (BT, BT), 1)
    striu = (rr < cc).astype(jnp.float32)
    cnt = cnt_scr[...]
    tot0 = jnp.sum(oh0, axis=1, keepdims=True)
    r0 = (jnp.sum(jax.lax.dot(oh0, striu,
                              preferred_element_type=jnp.float32) * oh0,
                  axis=0, keepdims=True)
          + jnp.sum(cnt * oh0, axis=0, keepdims=True))
    r1 = (jnp.sum(jax.lax.dot(oh1, striu,
                              preferred_element_type=jnp.float32) * oh1,
                  axis=0, keepdims=True)
          + jnp.sum((cnt + tot0) * oh1, axis=0, keepdims=True))
    rk0_ref[...] = r0.astype(jnp.int32)
    rk1_ref[...] = r1.astype(jnp.int32)
    cnt_scr[...] = cnt + tot0 + jnp.sum(oh1, axis=1, keepdims=True)
    cnt_ref[...] = cnt_scr[...]


def _router(flat, Wr, Wg_s, Wu_s, Wd_s):
    row_spec = pl.BlockSpec((1, BT), lambda t: (0, t))
    row_shape_i = jax.ShapeDtypeStruct((1, N), jnp.int32)
    row_shape_f = jax.ShapeDtypeStruct((1, N), jnp.float32)
    return pl.pallas_call(
        _router_body,
        grid=(N // BT,),
        in_specs=[
            pl.BlockSpec((BT, D), lambda t: (t, 0)),
            pl.BlockSpec((D, E), lambda t: (0, 0)),
            pl.BlockSpec((D, H), lambda t: (0, 0)),
            pl.BlockSpec((D, H), lambda t: (0, 0)),
            pl.BlockSpec((H, D), lambda t: (0, 0)),
        ],
        out_specs=[
            row_spec, row_spec, row_spec, row_spec, row_spec, row_spec,
            pl.BlockSpec((E, 1), lambda t: (0, 0)),
            pl.BlockSpec((BT, D), lambda t: (t, 0)),
            pl.BlockSpec((BT, D), lambda t: (t, 0)),
        ],
        out_shape=[
            row_shape_i, row_shape_i, row_shape_i, row_shape_i,
            row_shape_f, row_shape_f,
            jax.ShapeDtypeStruct((E, 1), jnp.float32),
            jax.ShapeDtypeStruct((N, D), jnp.bfloat16),
            jax.ShapeDtypeStruct((N, D), jnp.float32),
        ],
        scratch_shapes=[pltpu.VMEM((E, 1), jnp.float32)],
        compiler_params=pltpu.CompilerParams(
            dimension_semantics=("arbitrary",),
        ),
    )(flat, Wr, Wg_s, Wu_s, Wd_s)


def _finalize_body(cnt_ref, ti0_ref, ti1_ref, rk0_ref, rk1_ref,
                   p0_ref, p1_ref, be_ref, rb_ref, nb_ref):
    pci = (cnt_ref[...].astype(jnp.int32) + (BA - 1)) // BA
    ti0 = ti0_ref[...]
    ti1 = ti1_ref[...]
    acc0 = jnp.zeros((1, N), jnp.int32)
    acc1 = jnp.zeros((1, N), jnp.int32)
    bi = jax.lax.broadcasted_iota(jnp.int32, (1, NB), 1)
    be = jnp.zeros((1, NB), jnp.int32)
    bs = jnp.zeros((1, NB), jnp.int32)
    cum = jnp.zeros((1, 1), jnp.int32)
    for e in range(E):
        bstart_e = cum
        acc0 = acc0 + jnp.where(ti0 == e, bstart_e, 0)
        acc1 = acc1 + jnp.where(ti1 == e, bstart_e, 0)
        cum = cum + pci[e:e + 1, :]
        sel = bi >= cum
        be = be + jnp.where(sel & (be == e), 1, 0)
        bs = jnp.where(sel, cum, bs)
    be = jnp.minimum(be, E - 1)
    p0_ref[...] = acc0 * BA + rk0_ref[...]
    p1_ref[...] = acc1 * BA + rk1_ref[...]
    be_ref[...] = be
    rb_ref[...] = (bi - bs) * BA
    nb_ref[...] = cum


def _finalize(cnts, ti0, ti1, rk0, rk1):
    row_spec = pl.BlockSpec((1, N), lambda: (0, 0))
    return pl.pallas_call(
        _finalize_body,
        grid=(),
        in_specs=[pl.BlockSpec((E, 1), lambda: (0, 0)),
                  row_spec, row_spec, row_spec, row_spec],
        out_specs=[row_spec, row_spec,
                   pl.BlockSpec((1, NB), lambda: (0, 0)),
                   pl.BlockSpec((1, NB), lambda: (0, 0)),
                   pl.BlockSpec((1, 1), lambda: (0, 0))],
        out_shape=[
            jax.ShapeDtypeStruct((1, N), jnp.int32),
            jax.ShapeDtypeStruct((1, N), jnp.int32),
            jax.ShapeDtypeStruct((1, NB), jnp.int32),
            jax.ShapeDtypeStruct((1, NB), jnp.int32),
            jax.ShapeDtypeStruct((1, 1), jnp.int32),
        ],
    )(cnts, ti0, ti1, rk0, rk1)


def _ffn_body(be_ref, rb_ref, nb_ref, ti0_ref, ti1_ref, rk0_ref, rk1_ref,
              tp0_ref, tp1_ref, xb_ref, wg_ref, wu_ref, wd_ref, ys_ref):
    i = pl.program_id(0)

    @pl.when(i < nb_ref[0])
    def _():
        e = be_ref[i]
        rb = rb_ref[i]
        rows = jax.lax.broadcasted_iota(jnp.int32, (BA, N), 0) + rb
        c0 = (ti0_ref[...] == e) & (rk0_ref[...] == rows)
        c1 = (ti1_ref[...] == e) & (rk1_ref[...] == rows)
        oh = c0.astype(jnp.bfloat16) + c1.astype(jnp.bfloat16)
        xs = jax.lax.dot(oh, xb_ref[...],
                         preferred_element_type=jnp.float32)
        wgt = jnp.sum(jnp.where(c0, tp0_ref[...], 0.0)
                      + jnp.where(c1, tp1_ref[...], 0.0),
                      axis=1, keepdims=True)
        hg = _mm(xs, wg_ref[0])
        hu = _mm(xs, wu_ref[0])
        y = _mm(_silu(hg) * hu, wd_ref[0])
        ys_ref[...] = y * wgt


def _grouped_ffn(blk_exp, rbase, nb_used, meta_rows, xb16, Wg, Wu, Wd):
    grid_spec = pltpu.PrefetchScalarGridSpec(
        num_scalar_prefetch=3,
        grid=(NB,),
        in_specs=(
            [pl.BlockSpec((1, N), lambda i, be, rb, nb: (0, 0))] * 6
            + [
                pl.BlockSpec((N, D), lambda i, be, rb, nb: (0, 0)),
                pl.BlockSpec((1, D, H), lambda i, be, rb, nb: (be[i], 0, 0)),
                pl.BlockSpec((1, D, H), lambda i, be, rb, nb: (be[i], 0, 0)),
                pl.BlockSpec((1, H, D), lambda i, be, rb, nb: (be[i], 0, 0)),
            ]
        ),
        out_specs=pl.BlockSpec((BA, D), lambda i, be, rb, nb: (i, 0)),
    )
    return pl.pallas_call(
        _ffn_body,
        grid_spec=grid_spec,
        out_shape=jax.ShapeDtypeStruct((P, D), jnp.float32),
        compiler_params=pltpu.CompilerParams(
            dimension_semantics=("arbitrary",),
        ),
    )(blk_exp, rbase, nb_used, *meta_rows, xb16, Wg, Wu, Wd)


def _combine_body(p0_hbm, p1_hbm, sh_hbm, ys_hbm, out_hbm,
                  i0_v, i1_v, a_v, b_v, s_v, sem):
    wid = lax.axis_index("s") * NC + lax.axis_index("c")
    base = wid * TPW
    for c in range(TPW // CT):
        tb = base + c * CT
        pltpu.sync_copy(p0_hbm.at[pl.ds(tb, CT)], i0_v)
        pltpu.sync_copy(p1_hbm.at[pl.ds(tb, CT)], i1_v)
        cpa = pltpu.async_copy(ys_hbm.at[i0_v], a_v, sem)
        cpb = pltpu.async_copy(ys_hbm.at[i1_v], b_v, sem)
        pltpu.sync_copy(sh_hbm.at[pl.ds(tb, CT)], s_v)
        cpa.wait()
        cpb.wait()

        def _row(r, _):
            def _vec(j, _):
                sl = pl.ds(j * 16, 16)
                s_v[r, sl] = s_v[r, sl] + a_v[r, sl] + b_v[r, sl]
                return 0
            return lax.fori_loop(0, D // 16, _vec, 0, unroll=4)

        lax.fori_loop(0, CT, _row, 0)
        pltpu.sync_copy(s_v, out_hbm.at[pl.ds(tb, CT)])


def _combine(p0, p1, shared_out, ys):
    mesh = plsc.VectorSubcoreMesh(core_axis_name="c", subcore_axis_name="s")
    f = functools.partial(
        pl.kernel,
        mesh=mesh,
        out_type=jax.ShapeDtypeStruct((N, D), jnp.float32),
        scratch_types=[
            pltpu.VMEM((CT,), jnp.int32),
            pltpu.VMEM((CT,), jnp.int32),
            pltpu.VMEM((CT, D), jnp.float32),
            pltpu.VMEM((CT, D), jnp.float32),
            pltpu.VMEM((CT, D), jnp.float32),
            pltpu.SemaphoreType.DMA,
        ],
    )(_combine_body)
    return f(p0, p1, shared_out, ys)


def kernel(x, Wg_s, Wu_s, Wd_s, Wr, Wg, Wu, Wd):
    flat = x.reshape(N, D)
    (ti0, ti1, rk0, rk1, tp0, tp1, cnts, xb16,
     shared_out) = _router(flat, Wr, Wg_s, Wu_s, Wd_s)
    p0, p1, be, rb, nb = _finalize(cnts, ti0, ti1, rk0, rk1)
    meta_rows = (ti0, ti1, rk0, rk1, tp0, tp1)
    ys = _grouped_ffn(be.reshape(NB), rb.reshape(NB), nb.reshape(1),
                      meta_rows, xb16, Wg, Wu, Wd)
    out = _combine(p0.reshape(N), p1.reshape(N), shared_out, ys)
    return out.reshape(x.shape)
